# baseline (speedup 1.0000x reference)
import functools

import jax
import jax.numpy as jnp
from jax import lax
from jax.experimental import pallas as pl
from jax.experimental.pallas import tpu as pltpu
from jax.experimental.pallas import tpu_sc as plsc

BETA = 0.25
_KT = 1024

_NW = 32


def _make_sc_gather(v, d, b):
    b_per_w = b // _NW
    mesh = plsc.VectorSubcoreMesh(core_axis_name="c", subcore_axis_name="s")

    @functools.partial(
        pl.kernel, mesh=mesh,
        out_type=jax.ShapeDtypeStruct((b, d), jnp.float32),
        scratch_types=[
            pltpu.VMEM((b_per_w,), jnp.int32),
            pltpu.VMEM((b_per_w, d), jnp.float32),
            pltpu.SemaphoreType.DMA,
        ],
    )
    def gather_kernel(table_hbm, idx_hbm, out_hbm, idx_v, rows_v, sem):
        wid = lax.axis_index("s") * 2 + lax.axis_index("c")
        base = wid * b_per_w
        pltpu.sync_copy(idx_hbm.at[pl.ds(base, b_per_w)], idx_v)
        pltpu.async_copy(table_hbm.at[idx_v], rows_v, sem).wait()
        pltpu.sync_copy(rows_v, out_hbm.at[pl.ds(base, b_per_w)])

    return gather_kernel


_MT = 256


def _stage_body(rp_ref, zqp_ref, acc_ref, cb_ref, en_ref,
                idx_ref, r_ref, accn_ref, lp_ref, *, m, k, d):
    rp = rp_ref[...]
    zqp = zqp_ref[...]
    t = zqp - rp
    zq_st = rp + t
    r = rp - zq_st
    accn_ref[...] = acc_ref[...] + zq_st
    lp_ref[...] = jnp.sum(t * t).reshape(1, 1, 1)
    r_ref[...] = r

    rn = jnp.sum(r ** 2, axis=1, keepdims=True)
    r2 = r + r
    fiota = lax.broadcasted_iota(jnp.int32, (1, _KT), 1).astype(jnp.float32)
    nt = k // _KT

    mins = []
    args = []
    for kt in range(nt):
        cb_t = cb_ref[pl.ds(kt * _KT, _KT), :]
        en_t = en_ref[:, pl.ds(kt * _KT, _KT)]
        mm2 = lax.dot_general(
            r2, cb_t, (((1,), (1,)), ((), ())),
            preferred_element_type=jnp.float32)
        s = (rn + en_t) - mm2
        m_t = jnp.min(s, axis=1, keepdims=True)
        cand = jnp.where(s == m_t, fiota, jnp.float32(8192.0))
        i_t = jnp.min(cand, axis=1, keepdims=True) + jnp.float32(kt * _KT)
        mins.append(m_t)
        args.append(i_t)

    best_val, best_idx = mins[0], args[0]
    for kt in range(1, nt):
        upd = mins[kt] < best_val
        best_val = jnp.where(upd, mins[kt], best_val)
        best_idx = jnp.where(upd, args[kt], best_idx)

    idx_ref[...] = best_idx.astype(jnp.int32)


def _stage(rp, zqp, acc, cb, en):
    n, d = rp.shape
    k = cb.shape[0]
    body = functools.partial(_stage_body, m=_MT, k=k, d=d)
    return pl.pallas_call(
        body,
        grid=(n // _MT,),
        in_specs=[
            pl.BlockSpec((_MT, d), lambda i: (i, 0)),
            pl.BlockSpec((_MT, d), lambda i: (i, 0)),
            pl.BlockSpec((_MT, d), lambda i: (i, 0)),
            pl.BlockSpec((k, d), lambda i: (0, 0)),
            pl.BlockSpec((1, k), lambda i: (0, 0)),
        ],
        out_specs=(
            pl.BlockSpec((_MT, 1), lambda i: (i, 0)),
            pl.BlockSpec((_MT, d), lambda i: (i, 0)),
            pl.BlockSpec((_MT, d), lambda i: (i, 0)),
            pl.BlockSpec((1, 1, 1), lambda i: (i, 0, 0)),
        ),
        out_shape=(
            jax.ShapeDtypeStruct((n, 1), jnp.int32),
            jax.ShapeDtypeStruct((n, d), jnp.float32),
            jax.ShapeDtypeStruct((n, d), jnp.float32),
            jax.ShapeDtypeStruct((n // _MT, 1, 1), jnp.float32),
        ),
    )(rp, zqp, acc, cb, en)


def kernel(z, codebooks):
    B, T, D = z.shape
    zf = z.reshape(-1, D)
    n = zf.shape[0]
    en_all = jnp.sum(codebooks ** 2, axis=-1)
    nq = codebooks.shape[0]
    nelem = n * D

    n0 = (n // 2) // _MT * _MT
    bounds = [(0, n0), (n0, n)]
    gathers = [_make_sc_gather(codebooks.shape[1], D, hi - lo)
               for lo, hi in bounds]

    residual = [zf[lo:hi] for lo, hi in bounds]
    zq = [jnp.zeros((hi - lo, D), jnp.float32) for lo, hi in bounds]
    z_q = [jnp.zeros((hi - lo, D), jnp.float32) for lo, hi in bounds]
    lsums = [[], []]
    inds = []
    for q in range(nq):
        emb = codebooks[q]
        en = en_all[q].reshape(1, -1)
        idx_h = [None, None]
        for h in range(2):
            idx2, residual[h], z_q[h], lparts = _stage(
                residual[h], zq[h], z_q[h], emb, en)
            idx_h[h] = idx2.reshape(-1)
            zq[h] = gathers[h](emb, idx_h[h])
            if q > 0:
                lsums[h].append(jnp.sum(lparts))
        inds.append(jnp.concatenate(idx_h))
    losses = []
    last = []
    for h in range(2):
        t = zq[h] - residual[h]
        zq_st = residual[h] + t
        z_q[h] = z_q[h] + zq_st
        last.append(jnp.sum(t * t))
    for q in range(nq - 1):
        m = (lsums[0][q] + lsums[1][q]) / nelem
        losses.append(m + BETA * m)
    m = (last[0] + last[1]) / nelem
    losses.append(m + BETA * m)
    mean_losses = jnp.stack(losses).mean()
    all_min_encoding_indices = jnp.stack(inds, axis=1)
    z_q_full = jnp.concatenate(z_q, axis=0)
    return z_q_full.reshape(B, T, D), mean_losses, all_min_encoding_indices


# speedup vs baseline: 1.1269x; 1.1269x over previous
import functools

import jax
import jax.numpy as jnp
from jax import lax
from jax.experimental import pallas as pl
from jax.experimental.pallas import tpu as pltpu
from jax.experimental.pallas import tpu_sc as plsc

BETA = 0.25
_KT = 1024

_NW = 32


def _make_sc_gather(v, d, b):
    b_per_w = b // _NW
    mesh = plsc.VectorSubcoreMesh(core_axis_name="c", subcore_axis_name="s")

    @functools.partial(
        pl.kernel, mesh=mesh,
        out_type=jax.ShapeDtypeStruct((b, d), jnp.float32),
        scratch_types=[
            pltpu.VMEM((b_per_w,), jnp.int32),
            pltpu.VMEM((b_per_w, d), jnp.float32),
            pltpu.SemaphoreType.DMA,
        ],
    )
    def gather_kernel(table_hbm, idx_hbm, out_hbm, idx_v, rows_v, sem):
        wid = lax.axis_index("s") * 2 + lax.axis_index("c")
        base = wid * b_per_w
        pltpu.sync_copy(idx_hbm.at[pl.ds(base, b_per_w)], idx_v)
        pltpu.async_copy(table_hbm.at[idx_v], rows_v, sem).wait()
        pltpu.sync_copy(rows_v, out_hbm.at[pl.ds(base, b_per_w)])

    return gather_kernel


_MT = 1152


def _stage_body(rp_ref, zqp_ref, acc_ref, cb_ref, en_ref,
                idx_ref, r_ref, accn_ref, lp_ref, *, m, k, d):
    rp = rp_ref[...]
    zqp = zqp_ref[...]
    t = zqp - rp
    zq_st = rp + t
    r = rp - zq_st
    accn_ref[...] = acc_ref[...] + zq_st
    lp_ref[...] = jnp.sum(t * t).reshape(1, 1, 1)
    r_ref[...] = r

    rn = jnp.sum(r ** 2, axis=1, keepdims=True)
    r2 = r + r
    fiota = lax.broadcasted_iota(jnp.int32, (1, _KT), 1).astype(jnp.float32)
    nt = k // _KT

    mins = []
    args = []
    for kt in range(nt):
        cb_t = cb_ref[pl.ds(kt * _KT, _KT), :]
        en_t = en_ref[:, pl.ds(kt * _KT, _KT)]
        mm2 = lax.dot_general(
            r2, cb_t, (((1,), (1,)), ((), ())),
            preferred_element_type=jnp.float32)
        s = (rn + en_t) - mm2
        m_t = jnp.min(s, axis=1, keepdims=True)
        cand = jnp.where(s == m_t, fiota, jnp.float32(8192.0))
        i_t = jnp.min(cand, axis=1, keepdims=True) + jnp.float32(kt * _KT)
        mins.append(m_t)
        args.append(i_t)

    best_val, best_idx = mins[0], args[0]
    for kt in range(1, nt):
        upd = mins[kt] < best_val
        best_val = jnp.where(upd, mins[kt], best_val)
        best_idx = jnp.where(upd, args[kt], best_idx)

    idx_ref[...] = best_idx.astype(jnp.int32)


def _stage(rp, zqp, acc, cb, en):
    n, d = rp.shape
    k = cb.shape[0]
    body = functools.partial(_stage_body, m=_MT, k=k, d=d)
    return pl.pallas_call(
        body,
        grid=(n // _MT,),
        in_specs=[
            pl.BlockSpec((_MT, d), lambda i: (i, 0)),
            pl.BlockSpec((_MT, d), lambda i: (i, 0)),
            pl.BlockSpec((_MT, d), lambda i: (i, 0)),
            pl.BlockSpec((k, d), lambda i: (0, 0)),
            pl.BlockSpec((1, k), lambda i: (0, 0)),
        ],
        out_specs=(
            pl.BlockSpec((_MT, 1), lambda i: (i, 0)),
            pl.BlockSpec((_MT, d), lambda i: (i, 0)),
            pl.BlockSpec((_MT, d), lambda i: (i, 0)),
            pl.BlockSpec((1, 1, 1), lambda i: (i, 0, 0)),
        ),
        out_shape=(
            jax.ShapeDtypeStruct((n, 1), jnp.int32),
            jax.ShapeDtypeStruct((n, d), jnp.float32),
            jax.ShapeDtypeStruct((n, d), jnp.float32),
            jax.ShapeDtypeStruct((n // _MT, 1, 1), jnp.float32),
        ),
    )(rp, zqp, acc, cb, en)


def kernel(z, codebooks):
    B, T, D = z.shape
    zf = z.reshape(-1, D)
    n = zf.shape[0]
    en_all = jnp.sum(codebooks ** 2, axis=-1)
    nq = codebooks.shape[0]
    nelem = n * D

    sc_gather = _make_sc_gather(codebooks.shape[1], D, n)
    losses = []
    inds = []
    zeros = jnp.zeros_like(zf)
    residual, zq, z_q = zf, zeros, zeros
    for q in range(nq):
        emb = codebooks[q]
        idx2, residual, z_q, lparts = _stage(
            residual, zq, z_q, emb, en_all[q].reshape(1, -1))
        idx = idx2.reshape(-1)
        zq = sc_gather(emb, idx)
        if q > 0:
            m = jnp.sum(lparts) / nelem
            losses.append(m + BETA * m)
        inds.append(idx)
    t = zq - residual
    zq_st = residual + t
    z_q = z_q + zq_st
    m = jnp.sum(t * t) / nelem
    losses.append(m + BETA * m)
    mean_losses = jnp.stack(losses).mean()
    all_min_encoding_indices = jnp.stack(inds, axis=1)
    return z_q.reshape(B, T, D), mean_losses, all_min_encoding_indices
